# Initial kernel scaffold; baseline (speedup 1.0000x reference)
import jax
import jax.numpy as jnp
from jax.experimental import pallas as pl


def kernel(target_classes, prior_means, prior_logvars):
    raise NotImplementedError("write your pallas kernel here")


# speedup vs baseline: 1.4748x; 1.4748x over previous
import functools

import jax
import jax.numpy as jnp
from jax import lax
from jax.experimental import pallas as pl
from jax.experimental.pallas import tpu as pltpu
from jax.experimental.pallas import tpu_sc as plsc

LATENT = 128
BATCH = 16384
NC = 2
NS = 16
NW = NC * NS
B_PER_W = BATCH // NW
CHUNK = 128
NCHUNK = B_PER_W // CHUNK

_mesh = plsc.VectorSubcoreMesh(core_axis_name="c", subcore_axis_name="s")


@functools.partial(
    pl.kernel,
    mesh=_mesh,
    out_type=(
        jax.ShapeDtypeStruct((BATCH, LATENT), jnp.float32),
        jax.ShapeDtypeStruct((BATCH, LATENT), jnp.float32),
    ),
    scratch_types=[
        pltpu.VMEM((NCHUNK, CHUNK), jnp.int32),
        pltpu.VMEM((CHUNK, LATENT), jnp.float32),
        pltpu.VMEM((CHUNK, LATENT), jnp.float32),
        pltpu.SemaphoreType.DMA,
        pltpu.SemaphoreType.DMA,
    ],
)
def _gather2(idx_hbm, means_hbm, logvars_hbm, out_m, out_lv,
             idx_v, buf_m, buf_lv, sem_m, sem_lv):
    wid = lax.axis_index("s") * NC + lax.axis_index("c")
    pltpu.sync_copy(idx_hbm.at[wid], idx_v)
    base = wid * B_PER_W
    for c in range(NCHUNK):
        off = base + c * CHUNK
        gm = pltpu.async_copy(means_hbm.at[idx_v.at[c]], buf_m, sem_m)
        gl = pltpu.async_copy(logvars_hbm.at[idx_v.at[c]], buf_lv, sem_lv)
        gm.wait()
        pltpu.sync_copy(buf_m, out_m.at[pl.ds(off, CHUNK)])
        gl.wait()
        pltpu.sync_copy(buf_lv, out_lv.at[pl.ds(off, CHUNK)])


def kernel(target_classes, prior_means, prior_logvars):
    idx3 = target_classes.reshape(NW, NCHUNK, CHUNK)
    return _gather2(idx3, prior_means, prior_logvars)


# speedup vs baseline: 1.5579x; 1.0564x over previous
import functools

import jax
import jax.numpy as jnp
from jax import lax
from jax.experimental import pallas as pl
from jax.experimental.pallas import tpu as pltpu
from jax.experimental.pallas import tpu_sc as plsc

LATENT = 128
BATCH = 16384
NC = 2
NS = 16
NW = NC * NS
B_PER_W = BATCH // NW
CHUNK = 128
NCHUNK = B_PER_W // CHUNK

_mesh = plsc.VectorSubcoreMesh(core_axis_name="c", subcore_axis_name="s")


@functools.partial(
    pl.kernel,
    mesh=_mesh,
    out_type=(
        jax.ShapeDtypeStruct((BATCH, LATENT), jnp.float32),
        jax.ShapeDtypeStruct((BATCH, LATENT), jnp.float32),
    ),
    scratch_types=[
        pltpu.VMEM((NCHUNK, CHUNK), jnp.int32),
        pltpu.VMEM((CHUNK, LATENT), jnp.float32),
        pltpu.VMEM((CHUNK, LATENT), jnp.float32),
        pltpu.VMEM((CHUNK, LATENT), jnp.float32),
        pltpu.VMEM((CHUNK, LATENT), jnp.float32),
        pltpu.SemaphoreType.DMA,
        pltpu.SemaphoreType.DMA,
        pltpu.SemaphoreType.DMA,
        pltpu.SemaphoreType.DMA,
        pltpu.SemaphoreType.DMA,
        pltpu.SemaphoreType.DMA,
        pltpu.SemaphoreType.DMA,
        pltpu.SemaphoreType.DMA,
    ],
)
def _gather2(idx_hbm, means_hbm, logvars_hbm, out_m, out_lv,
             idx_v, bm0, bm1, bl0, bl1,
             sgm0, sgm1, sgl0, sgl1, som0, som1, sol0, sol1):
    wid = lax.axis_index("s") * NC + lax.axis_index("c")
    pltpu.sync_copy(idx_hbm.at[wid], idx_v)
    base = wid * B_PER_W
    bm, bl = (bm0, bm1), (bl0, bl1)
    sgm, sgl = (sgm0, sgm1), (sgl0, sgl1)
    som, sol = (som0, som1), (sol0, sol1)
    gm_d = [None] * NCHUNK
    gl_d = [None] * NCHUNK
    om_d = [None] * NCHUNK
    ol_d = [None] * NCHUNK
    for c in range(NCHUNK + 1):
        if c < NCHUNK:
            s = c % 2
            if c >= 2:
                om_d[c - 2].wait()
                ol_d[c - 2].wait()
            gm_d[c] = pltpu.async_copy(means_hbm.at[idx_v.at[c]], bm[s], sgm[s])
            gl_d[c] = pltpu.async_copy(logvars_hbm.at[idx_v.at[c]], bl[s], sgl[s])
        if c >= 1:
            p = c - 1
            s = p % 2
            off = base + p * CHUNK
            gm_d[p].wait()
            om_d[p] = pltpu.async_copy(bm[s], out_m.at[pl.ds(off, CHUNK)], som[s])
            gl_d[p].wait()
            ol_d[p] = pltpu.async_copy(bl[s], out_lv.at[pl.ds(off, CHUNK)], sol[s])
    om_d[NCHUNK - 1].wait()
    ol_d[NCHUNK - 1].wait()


def kernel(target_classes, prior_means, prior_logvars):
    idx3 = target_classes.reshape(NW, NCHUNK, CHUNK)
    return _gather2(idx3, prior_means, prior_logvars)
